# Initial kernel scaffold; baseline (speedup 1.0000x reference)
import jax
import jax.numpy as jnp
from jax.experimental import pallas as pl


def kernel(query_points, ref_points, ref_flow, k):
    raise NotImplementedError("write your pallas kernel here")


# speedup vs baseline: 4.9150x; 4.9150x over previous
import jax
import jax.numpy as jnp
from jax.experimental import pallas as pl

_N = 10000
_M = 10000
_BQ = 200
_M_PAD = 10112
_FAR = 1e9


def _knn_flow_kernel(q_ref, rpt_ref, rft_ref, out_ref):
    q = q_ref[...]
    qx, qy, qz = q[:, 0:1], q[:, 1:2], q[:, 2:3]
    rx, ry, rz = rpt_ref[0:1, :], rpt_ref[1:2, :], rpt_ref[2:3, :]

    def _b(x):
        return x.astype(jnp.bfloat16).astype(jnp.float32)

    q2 = qx * qx + qy * qy + qz * qz
    r2 = rx * rx + ry * ry + rz * rz
    qr = _b(qx) * _b(rx) + _b(qy) * _b(ry) + _b(qz) * _b(rz)
    d2 = (q2 - 2.0 * qr) + r2
    dist = jnp.sqrt(jnp.maximum(d2, 0.0))

    big = jnp.float32(jnp.inf)
    bigidx = jnp.float32(1e9)
    iota = jax.lax.broadcasted_iota(jnp.int32, (1, _M_PAD), 1).astype(jnp.float32)

    dcur = dist
    ms, idxs = [], []
    for _ in range(3):
        m = jnp.min(dcur, axis=1, keepdims=True)
        i = jnp.min(jnp.where(dcur == m, iota, bigidx), axis=1, keepdims=True)
        dcur = jnp.where(iota == i, big, dcur)
        ms.append(m)
        idxs.append(i)

    inv1 = 1.0 / (ms[0] + 1e-8)
    inv2 = 1.0 / (ms[1] + 1e-8)
    inv3 = 1.0 / (ms[2] + 1e-8)
    zero = jnp.float32(0.0)
    w = (jnp.where(iota == idxs[0], inv1, zero)
         + jnp.where(iota == idxs[1], inv2, zero)
         + jnp.where(iota == idxs[2], inv3, zero))
    sw = inv1 + inv2 + inv3
    fx = jnp.sum(w * rft_ref[0:1, :], axis=1, keepdims=True)
    fy = jnp.sum(w * rft_ref[1:2, :], axis=1, keepdims=True)
    fz = jnp.sum(w * rft_ref[2:3, :], axis=1, keepdims=True)
    out_ref[...] = jnp.concatenate([fx, fy, fz], axis=1) / sw


def kernel(query_points, ref_points, ref_flow, k):
    del k
    pad = _M_PAD - _M
    rpt = jnp.pad(ref_points, ((0, pad), (0, 0)), constant_values=_FAR).T
    rft = jnp.pad(ref_flow, ((0, pad), (0, 0))).T

    grid = _N // _BQ
    out = pl.pallas_call(
        _knn_flow_kernel,
        grid=(grid,),
        in_specs=[
            pl.BlockSpec((_BQ, 3), lambda b: (b, 0)),
            pl.BlockSpec((3, _M_PAD), lambda b: (0, 0)),
            pl.BlockSpec((3, _M_PAD), lambda b: (0, 0)),
        ],
        out_specs=pl.BlockSpec((_BQ, 3), lambda b: (b, 0)),
        out_shape=jax.ShapeDtypeStruct((_N, 3), jnp.float32),
    )(query_points, rpt, rft)
    return out


# speedup vs baseline: 6.7487x; 1.3731x over previous
import jax
import jax.numpy as jnp
from jax.experimental import pallas as pl

_N = 10000
_M = 10000
_BQ = 200
_M_PAD = 10112
_FAR = 1e9


def _knn_flow_kernel(q_ref, rpt_ref, rb_ref, rft_ref, out_ref):
    q = q_ref[...]
    qx, qy, qz = q[:, 0:1], q[:, 1:2], q[:, 2:3]
    rx, ry, rz = rpt_ref[0:1, :], rpt_ref[1:2, :], rpt_ref[2:3, :]

    q2 = qx * qx + qy * qy + qz * qz
    r2 = rx * rx + ry * ry + rz * rz
    qrm2 = jax.lax.dot_general(
        (q * -2.0).astype(jnp.bfloat16), rb_ref[...],
        dimension_numbers=(((1,), (0,)), ((), ())),
        preferred_element_type=jnp.float32,
    )
    s = jnp.maximum((q2 + qrm2) + r2, 0.0)

    big = jnp.float32(jnp.inf)
    bigidx = jnp.float32(1e9)
    iota = jax.lax.broadcasted_iota(jnp.int32, (1, _M_PAD), 1).astype(jnp.float32)

    scur = s
    ms, idxs = [], []
    for _ in range(3):
        m = jnp.min(scur, axis=1, keepdims=True)
        i = jnp.min(jnp.where(scur == m, iota, bigidx), axis=1, keepdims=True)
        scur = jnp.where(iota == i, big, scur)
        ms.append(m)
        idxs.append(i)

    inv1 = 1.0 / (jnp.sqrt(ms[0]) + 1e-8)
    inv2 = 1.0 / (jnp.sqrt(ms[1]) + 1e-8)
    inv3 = 1.0 / (jnp.sqrt(ms[2]) + 1e-8)
    zero = jnp.float32(0.0)
    w = (jnp.where(iota == idxs[0], inv1, zero)
         + jnp.where(iota == idxs[1], inv2, zero)
         + jnp.where(iota == idxs[2], inv3, zero))
    sw = inv1 + inv2 + inv3
    fx = jnp.sum(w * rft_ref[0:1, :], axis=1, keepdims=True)
    fy = jnp.sum(w * rft_ref[1:2, :], axis=1, keepdims=True)
    fz = jnp.sum(w * rft_ref[2:3, :], axis=1, keepdims=True)
    out_ref[...] = jnp.concatenate([fx, fy, fz], axis=1) / sw


def kernel(query_points, ref_points, ref_flow, k):
    del k
    pad = _M_PAD - _M
    rpt = jnp.pad(ref_points, ((0, pad), (0, 0)), constant_values=_FAR).T
    rft = jnp.pad(ref_flow, ((0, pad), (0, 0))).T
    rb = rpt.astype(jnp.bfloat16)

    grid = _N // _BQ
    out = pl.pallas_call(
        _knn_flow_kernel,
        grid=(grid,),
        in_specs=[
            pl.BlockSpec((_BQ, 3), lambda b: (b, 0)),
            pl.BlockSpec((3, _M_PAD), lambda b: (0, 0)),
            pl.BlockSpec((3, _M_PAD), lambda b: (0, 0)),
            pl.BlockSpec((3, _M_PAD), lambda b: (0, 0)),
        ],
        out_specs=pl.BlockSpec((_BQ, 3), lambda b: (b, 0)),
        out_shape=jax.ShapeDtypeStruct((_N, 3), jnp.float32),
    )(query_points, rpt, rb, rft)
    return out


# speedup vs baseline: 8.7950x; 1.3032x over previous
import jax
import jax.numpy as jnp
from jax.experimental import pallas as pl

_N = 10000
_M = 10000
_BQ = 200
_M_PAD = 10112
_FAR = 1e9


def _knn_flow_kernel(q_ref, rpt_ref, rb_ref, rft_ref, out_ref):
    q = q_ref[...]
    qx, qy, qz = q[:, 0:1], q[:, 1:2], q[:, 2:3]
    rx, ry, rz = rpt_ref[0:1, :], rpt_ref[1:2, :], rpt_ref[2:3, :]

    q2 = qx * qx + qy * qy + qz * qz
    r2 = rx * rx + ry * ry + rz * rz
    qrm2 = jax.lax.dot_general(
        (q * -2.0).astype(jnp.bfloat16), rb_ref[...],
        dimension_numbers=(((1,), (0,)), ((), ())),
        preferred_element_type=jnp.float32,
    )
    s = jnp.maximum((q2 + qrm2) + r2, 0.0)

    big = jnp.float32(jnp.inf)
    bigidx = jnp.float32(1e9)
    iota = jax.lax.broadcasted_iota(jnp.int32, (1, _M_PAD), 1).astype(jnp.float32)

    m1 = jnp.min(s, axis=1, keepdims=True)
    i1 = jnp.min(jnp.where(s == m1, iota, bigidx), axis=1, keepdims=True)
    eq1 = iota == i1
    s2 = jnp.where(eq1, big, s)
    m2 = jnp.min(s2, axis=1, keepdims=True)
    i2 = jnp.min(jnp.where(s2 == m2, iota, bigidx), axis=1, keepdims=True)
    eq2 = iota == i2
    s3 = jnp.where(eq2, big, s2)
    m3 = jnp.min(s3, axis=1, keepdims=True)
    i3 = jnp.min(jnp.where(s3 == m3, iota, bigidx), axis=1, keepdims=True)
    eq3 = iota == i3

    def _b16(x):
        return x.astype(jnp.bfloat16).astype(jnp.float32)

    inv1 = _b16(1.0 / (jnp.sqrt(m1) + 1e-8))
    inv2 = _b16(1.0 / (jnp.sqrt(m2) + 1e-8))
    inv3 = _b16(1.0 / (jnp.sqrt(m3) + 1e-8))
    zero = jnp.float32(0.0)
    w = jnp.where(eq1, inv1, jnp.where(eq2, inv2, jnp.where(eq3, inv3, zero)))
    sw = inv1 + inv2 + inv3
    flow3 = jax.lax.dot_general(
        w.astype(jnp.bfloat16), rft_ref[...],
        dimension_numbers=(((1,), (0,)), ((), ())),
        preferred_element_type=jnp.float32,
    )
    out_ref[...] = flow3 / sw


def kernel(query_points, ref_points, ref_flow, k):
    del k
    pad = _M_PAD - _M
    rpt = jnp.pad(ref_points, ((0, pad), (0, 0)), constant_values=_FAR).T
    rftb = jnp.pad(ref_flow, ((0, pad), (0, 0))).astype(jnp.bfloat16)
    rb = rpt.astype(jnp.bfloat16)

    grid = _N // _BQ
    out = pl.pallas_call(
        _knn_flow_kernel,
        grid=(grid,),
        in_specs=[
            pl.BlockSpec((_BQ, 3), lambda b: (b, 0)),
            pl.BlockSpec((3, _M_PAD), lambda b: (0, 0)),
            pl.BlockSpec((3, _M_PAD), lambda b: (0, 0)),
            pl.BlockSpec((_M_PAD, 3), lambda b: (0, 0)),
        ],
        out_specs=pl.BlockSpec((_BQ, 3), lambda b: (b, 0)),
        out_shape=jax.ShapeDtypeStruct((_N, 3), jnp.float32),
    )(query_points, rpt, rb, rftb)
    return out


# speedup vs baseline: 11.2304x; 1.2769x over previous
import jax
import jax.numpy as jnp
from jax.experimental import pallas as pl

_N = 10000
_M = 10000
_BQ = 200
_M_PAD = 10112
_FAR = 1e9


def _knn_flow_kernel(q_ref, rpt_ref, rb_ref, rft_ref, out_ref):
    q = q_ref[...]
    qx, qy, qz = q[:, 0:1], q[:, 1:2], q[:, 2:3]
    rx, ry, rz = rpt_ref[0:1, :], rpt_ref[1:2, :], rpt_ref[2:3, :]

    q2 = qx * qx + qy * qy + qz * qz
    r2 = rx * rx + ry * ry + rz * rz
    qrm2 = jax.lax.dot_general(
        (q * -2.0).astype(jnp.bfloat16), rb_ref[...],
        dimension_numbers=(((1,), (0,)), ((), ())),
        preferred_element_type=jnp.float32,
    )
    s = jnp.maximum((q2 + qrm2) + r2, 0.0)

    big = jnp.float32(jnp.inf)
    bigidx = jnp.float32(1e9)
    iota = jax.lax.broadcasted_iota(jnp.int32, (1, _M_PAD), 1).astype(jnp.float32)

    m1 = jnp.min(s, axis=1, keepdims=True)
    i1 = jnp.min(jnp.where(s == m1, iota, bigidx), axis=1, keepdims=True)
    s2 = jnp.where(iota == i1, big, s)
    m2 = jnp.min(s2, axis=1, keepdims=True)
    i2 = jnp.min(jnp.where(s2 == m2, iota, bigidx), axis=1, keepdims=True)
    s3 = jnp.where(iota == i2, big, s2)
    m3 = jnp.min(s3, axis=1, keepdims=True)
    i3 = jnp.min(jnp.where(s3 == m3, iota, bigidx), axis=1, keepdims=True)

    iota128 = jax.lax.broadcasted_iota(jnp.int32, (1, 128), 1).astype(jnp.float32)
    rfa = rft_ref[...]
    zero = jnp.float32(0.0)

    def _pick(i):
        g = jnp.floor(i * (1.0 / 128.0))
        l = i - g * 128.0
        oh = jnp.where(iota128 == g, 1.0, zero)
        p = jax.lax.dot_general(
            oh, rfa, dimension_numbers=(((1,), (0,)), ((), ())),
            preferred_element_type=jnp.float32,
        )
        eql = iota128 == l
        return p, eql

    p1, el1 = _pick(i1)
    p2, el2 = _pick(i2)
    p3, el3 = _pick(i3)

    inv1 = 1.0 / (jnp.sqrt(m1) + 1e-8)
    inv2 = 1.0 / (jnp.sqrt(m2) + 1e-8)
    inv3 = 1.0 / (jnp.sqrt(m3) + 1e-8)
    sw = inv1 + inv2 + inv3

    def _comp(c):
        lo, hi = c * 128, (c + 1) * 128
        acc = (inv1 * jnp.where(el1, p1[:, lo:hi], zero)
               + inv2 * jnp.where(el2, p2[:, lo:hi], zero)
               + inv3 * jnp.where(el3, p3[:, lo:hi], zero))
        return jnp.sum(acc, axis=1, keepdims=True)

    out_ref[...] = jnp.concatenate([_comp(0), _comp(1), _comp(2)], axis=1) / sw


def kernel(query_points, ref_points, ref_flow, k):
    del k
    pad = _M_PAD - _M
    rpt = jnp.pad(ref_points, ((0, pad), (0, 0)), constant_values=_FAR).T
    rfa = (jnp.pad(ref_flow, ((0, pad), (0, 0)))
           .reshape(_M_PAD // 128, 128, 3)
           .transpose(0, 2, 1)
           .reshape(_M_PAD // 128, 384))
    rfa = jnp.pad(rfa, ((0, 128 - _M_PAD // 128), (0, 0)))
    rb = rpt.astype(jnp.bfloat16)

    grid = _N // _BQ
    out = pl.pallas_call(
        _knn_flow_kernel,
        grid=(grid,),
        in_specs=[
            pl.BlockSpec((_BQ, 3), lambda b: (b, 0)),
            pl.BlockSpec((3, _M_PAD), lambda b: (0, 0)),
            pl.BlockSpec((3, _M_PAD), lambda b: (0, 0)),
            pl.BlockSpec((128, 384), lambda b: (0, 0)),
        ],
        out_specs=pl.BlockSpec((_BQ, 3), lambda b: (b, 0)),
        out_shape=jax.ShapeDtypeStruct((_N, 3), jnp.float32),
    )(query_points, rpt, rb, rfa)
    return out
